# baseline (speedup 1.0000x reference)
import jax
import jax.numpy as jnp
from jax import lax
from jax.experimental import pallas as pl
from jax.experimental.pallas import tpu as pltpu
from jax.experimental.pallas import tpu_sc as plsc

_SIZE = 1.0
_RES = 64
_EMBED = 16
_TABLE = 2 ** 21
_N = 1048576

_NC = 2
_NS = 16
_NW = _NC * _NS

_PER_W = _N // _NW
_CHUNK = 2048
_NCHUNK = _PER_W // _CHUNK
_GSUB = 128
_NG = _CHUNK // _GSUB
_CBLK = _CHUNK // 128
_OBLK_W = _PER_W // 128

_NBLK = _TABLE // 128
_BLK_W = _NBLK // _NW
_RB = 8
_RCH = _BLK_W // _RB
_RENT = _RB * 128

_P2 = 2654435761
_P3 = 805459861


def _hash_tc_body(x_ref, y_ref, z_ref, o_ref):
    def quant(v):
        f = (v / _SIZE + 0.5) * _RES
        return jnp.minimum(f.astype(jnp.int32), _RES - 1)

    cx = quant(x_ref[...])
    cy = quant(y_ref[...])
    cz = quant(z_ref[...])
    h = cx ^ (cy * jnp.int32(_P2 - 2 ** 32)) ^ (cz * jnp.int32(_P3))
    o_ref[...] = h & jnp.int32(_TABLE - 1)


def _relayout_body(nt_hbm, out_hbm, rows0, rows1, sem_in, sem_out):
    wid = lax.axis_index("s") * _NC + lax.axis_index("c")
    rows = (rows0, rows1)

    def in_descs(par, c):
        b0 = wid * _BLK_W + c * _RB
        ds_ = []
        for blk in range(_RB):
            for e in range(2):
                for fl in range(8):
                    ds_.append(pltpu.make_async_copy(
                        nt_hbm.at[e, b0 + blk, fl],
                        rows[par].at[pl.ds(blk * 128, 128),
                                     pl.ds(8 * e + fl, 1)],
                        sem_in))
        return ds_

    def out_desc(par, c):
        ent0 = (wid * _BLK_W + c * _RB) * 128
        return pltpu.make_async_copy(
            rows[par], out_hbm.at[pl.ds(ent0, _RENT)], sem_out)

    for d in in_descs(0, 0):
        d.start()

    @pl.loop(0, _RCH, step=2)
    def outer(i):
        for par in range(2):
            c = i + par
            for d in in_descs(par, c):
                d.wait()
            out_desc(par, c).start()

            @pl.when(c >= 1)
            def _():
                out_desc(1 - par, c - 1).wait()

            @pl.when(c + 1 < _RCH)
            def _():
                for d in in_descs(1 - par, c + 1):
                    d.start()

    out_desc((_RCH - 1) % 2, _RCH - 1).wait()


def _gather_body(idx_hbm, table_hbm, out_hbm,
                 idx0, idx1, rowsv0, rowsv1, gsem0, gsem1, osem0, osem1):
    wid = lax.axis_index("s") * _NC + lax.axis_index("c")
    base = wid * _PER_W
    idxs = (idx0, idx1)
    rowsv = (rowsv0, rowsv1)
    gsems = (gsem0, gsem1)
    osems = (osem0, osem1)

    def g_descs(par):
        return [pltpu.make_async_copy(
            table_hbm.at[idxs[par].at[pl.ds(g * _GSUB, _GSUB)]],
            rowsv[par].at[pl.ds(g * _GSUB, _GSUB)],
            gsems[par]) for g in range(_NG)]

    def o_descs(par, c):
        blk0 = wid * _OBLK_W + c * _CBLK
        ds_ = []
        for blk in range(_CBLK):
            for e in range(2):
                for fl in range(8):
                    ds_.append(pltpu.make_async_copy(
                        rowsv[par].at[pl.ds(blk * 128, 128),
                                      pl.ds(8 * e + fl, 1)],
                        out_hbm.at[e, blk0 + blk, fl],
                        osems[par]))
        return ds_

    pltpu.sync_copy(idx_hbm.at[pl.ds(base, _CHUNK)], idx0)
    for d in g_descs(0):
        d.start()

    @pl.loop(0, _NCHUNK, step=2)
    def outer(i):
        for par in range(2):
            c = i + par
            for d in g_descs(par):
                d.wait()
            for d in o_descs(par, c):
                d.start()

            @pl.when(c >= 1)
            def _():
                for d in o_descs(1 - par, c - 1):
                    d.wait()

            @pl.when(c + 1 < _NCHUNK)
            def _():
                pltpu.sync_copy(
                    idx_hbm.at[pl.ds(base + (c + 1) * _CHUNK, _CHUNK)],
                    idxs[1 - par])
                for d in g_descs(1 - par):
                    d.start()

    for d in o_descs((_NCHUNK - 1) % 2, _NCHUNK - 1):
        d.wait()


_SC_PARAMS = pltpu.CompilerParams(
    needs_layout_passes=False, use_tc_tiling_on_sc=False,
    disable_bounds_checks=True)


@jax.jit
def _lookup(xyz, table):
    x = xyz[:, 0].reshape(_N // 128, 128)
    y = xyz[:, 1].reshape(_N // 128, 128)
    z = xyz[:, 2].reshape(_N // 128, 128)

    idx2d = pl.pallas_call(
        _hash_tc_body,
        out_shape=jax.ShapeDtypeStruct((_N // 128, 128), jnp.int32),
    )(x, y, z)
    idx_flat = idx2d.reshape(-1)

    nt = (table.T.reshape(2, 8, _NBLK, 128)
          .transpose(0, 2, 1, 3).reshape(2, _NBLK, 8, 128, 1))

    mesh = plsc.VectorSubcoreMesh(
        core_axis_name="c", subcore_axis_name="s",
        num_cores=_NC, num_subcores=_NS,
    )
    table_rm = pl.kernel(
        _relayout_body,
        out_type=jax.ShapeDtypeStruct((_TABLE, _EMBED), jnp.float32),
        mesh=mesh,
        scratch_types=[
            pltpu.VMEM((_RENT, _EMBED), jnp.float32),
            pltpu.VMEM((_RENT, _EMBED), jnp.float32),
            pltpu.SemaphoreType.DMA,
            pltpu.SemaphoreType.DMA,
        ],
        compiler_params=_SC_PARAMS,
    )(nt)

    out5 = pl.kernel(
        _gather_body,
        out_type=jax.ShapeDtypeStruct((2, _N // 128, 8, 128, 1), jnp.float32),
        mesh=mesh,
        scratch_types=[
            pltpu.VMEM((_CHUNK,), jnp.int32),
            pltpu.VMEM((_CHUNK,), jnp.int32),
            pltpu.VMEM((_CHUNK, _EMBED), jnp.float32),
            pltpu.VMEM((_CHUNK, _EMBED), jnp.float32),
            pltpu.SemaphoreType.DMA,
            pltpu.SemaphoreType.DMA,
            pltpu.SemaphoreType.DMA,
            pltpu.SemaphoreType.DMA,
        ],
        compiler_params=_SC_PARAMS,
    )(idx_flat, table_rm)

    return (out5.reshape(2, _N // 128, 8, 128)
            .transpose(1, 3, 0, 2).reshape(_N, _EMBED))


def kernel(xyz, table):
    return _lookup(xyz, table)


# speedup vs baseline: 150.2956x; 150.2956x over previous
import jax
import jax.numpy as jnp
from jax import lax
from jax.experimental import pallas as pl
from jax.experimental.pallas import tpu as pltpu
from jax.experimental.pallas import tpu_sc as plsc

_SIZE = 1.0
_RES = 64
_EMBED = 16
_TABLE = 2 ** 21
_N = 1048576

_NC = 2
_NS = 16
_NW = _NC * _NS
_L = 16

_PER_W = _N // _NW
_CHUNK = 2048
_NCHUNK = _PER_W // _CHUNK
_GSUB = 128
_NG = _CHUNK // _GSUB
_OBLK_W = _PER_W // 128

_NBLK = _TABLE // 128
_BLK_W = _NBLK // _NW
_RB = 8
_RCH = _BLK_W // _RB
_RENT = _RB * 128

_P2 = 2654435761
_P3 = 805459861


def _hash_tc_body(x_ref, y_ref, z_ref, o_ref):
    def quant(v):
        f = (v / _SIZE + 0.5) * _RES
        return jnp.minimum(f.astype(jnp.int32), _RES - 1)

    cx = quant(x_ref[...])
    cy = quant(y_ref[...])
    cz = quant(z_ref[...])
    h = cx ^ (cy * jnp.int32(_P2 - 2 ** 32)) ^ (cz * jnp.int32(_P3))
    o_ref[...] = h & jnp.int32(_TABLE - 1)


def _relayout_body(nt_hbm, out_hbm, buf0, buf1, rows0, rows1,
                   sem_in, sem_out):
    wid = lax.axis_index("s") * _NC + lax.axis_index("c")
    lane16 = lax.iota(jnp.int32, _L) * 16
    bufs = (buf0, buf1)
    rows = (rows0, rows1)

    def in_descs(par, c):
        b0 = wid * _BLK_W + c * _RB
        return [
            pltpu.make_async_copy(nt_hbm.at[e, pl.ds(b0, _RB)],
                                  bufs[par].at[e], sem_in)
            for e in range(2)
        ]

    def out_desc(par, c):
        ent0 = (wid * _BLK_W + c * _RB) * 128
        return pltpu.make_async_copy(
            rows[par], out_hbm.at[pl.ds(ent0 * _EMBED, _RENT * _EMBED)],
            sem_out)

    for d in in_descs(0, 0):
        d.start()

    @pl.loop(0, _RCH, step=2)
    def outer(i):
        for par in range(2):
            c = i + par
            for d in in_descs(par, c):
                d.wait()

            @pl.when(c + 1 < _RCH)
            def _():
                for d in in_descs(1 - par, c + 1):
                    d.start()

            @pl.when(c >= 2)
            def _():
                out_desc(par, c - 2).wait()

            for blk in range(_RB):
                vb = lane16 + blk * (128 * _EMBED)
                for e in range(2):
                    for fl in range(8):
                        f = 8 * e + fl
                        vals = [bufs[par][e, blk, fl, pl.ds(g * _L, _L)]
                                for g in range(8)]
                        for g in range(8):
                            plsc.store_scatter(
                                rows[par], [vb + (g * _L * _EMBED + f)],
                                vals[g])
            out_desc(par, c).start()

    for c in (_RCH - 2, _RCH - 1):
        out_desc(c % 2, c).wait()


def _gather_body(idx_hbm, table_hbm, out_hbm, idx_v, rows_v, tr_v, gsem):
    wid = lax.axis_index("s") * _NC + lax.axis_index("c")
    base = wid * _PER_W
    lane = lax.iota(jnp.int32, _L)
    colf = [jnp.full((_L,), f, jnp.int32) for f in range(_EMBED)]

    def chunk_body(c, carry):
        pstart = base + c * _CHUNK
        pltpu.sync_copy(idx_hbm.at[pl.ds(pstart, _CHUNK)], idx_v)
        descs = []
        for g in range(_NG):
            descs.append(
                pltpu.async_copy(
                    table_hbm.at[idx_v.at[pl.ds(g * _GSUB, _GSUB)]],
                    rows_v.at[pl.ds(g * _GSUB, _GSUB)],
                    gsem,
                )
            )
        for d in descs:
            d.wait()

        for blk in range(_CHUNK // 128):
            vb = lane + blk * 128
            for e in range(2):
                for fl in range(8):
                    f = 8 * e + fl
                    vals = [plsc.load_gather(rows_v, [vb + g * _L, colf[f]])
                            for g in range(8)]
                    for g in range(8):
                        tr_v[e, blk, fl, pl.ds(g * _L, _L)] = vals[g]

        blk0 = wid * _OBLK_W + c * (_CHUNK // 128)
        for e in range(2):
            pltpu.sync_copy(tr_v.at[e],
                            out_hbm.at[e, pl.ds(blk0, _CHUNK // 128)])
        return carry

    lax.fori_loop(0, _NCHUNK, chunk_body, 0)


_SC_PARAMS = pltpu.CompilerParams(
    needs_layout_passes=False, use_tc_tiling_on_sc=False,
    disable_bounds_checks=True)


@jax.jit
def _lookup(xyz, table):
    x = xyz[:, 0].reshape(_N // 128, 128)
    y = xyz[:, 1].reshape(_N // 128, 128)
    z = xyz[:, 2].reshape(_N // 128, 128)

    idx2d = pl.pallas_call(
        _hash_tc_body,
        out_shape=jax.ShapeDtypeStruct((_N // 128, 128), jnp.int32),
    )(x, y, z)
    idx_flat = idx2d.reshape(-1)

    nt = table.T.reshape(2, 8, _NBLK, 128).transpose(0, 2, 1, 3)

    mesh = plsc.VectorSubcoreMesh(
        core_axis_name="c", subcore_axis_name="s",
        num_cores=_NC, num_subcores=_NS,
    )
    table_rm_flat = pl.kernel(
        _relayout_body,
        out_type=jax.ShapeDtypeStruct((_TABLE * _EMBED,), jnp.float32),
        mesh=mesh,
        scratch_types=[
            pltpu.VMEM((2, _RB, 8, 128), jnp.float32),
            pltpu.VMEM((2, _RB, 8, 128), jnp.float32),
            pltpu.VMEM((_RENT * _EMBED,), jnp.float32),
            pltpu.VMEM((_RENT * _EMBED,), jnp.float32),
            pltpu.SemaphoreType.DMA,
            pltpu.SemaphoreType.DMA,
        ],
        compiler_params=_SC_PARAMS,
    )(nt)
    table_rm = table_rm_flat.reshape(_TABLE, _EMBED)

    out4 = pl.kernel(
        _gather_body,
        out_type=jax.ShapeDtypeStruct((2, _N // 128, 8, 128), jnp.float32),
        mesh=mesh,
        scratch_types=[
            pltpu.VMEM((_CHUNK,), jnp.int32),
            pltpu.VMEM((_CHUNK, _EMBED), jnp.float32),
            pltpu.VMEM((2, _CHUNK // 128, 8, 128), jnp.float32),
            pltpu.SemaphoreType.DMA,
        ],
        compiler_params=_SC_PARAMS,
    )(idx_flat, table_rm)

    return out4.transpose(1, 3, 0, 2).reshape(_N, _EMBED)


def kernel(xyz, table):
    return _lookup(xyz, table)


# speedup vs baseline: 157.0534x; 1.0450x over previous
import jax
import jax.numpy as jnp
from jax import lax
from jax.experimental import pallas as pl
from jax.experimental.pallas import tpu as pltpu
from jax.experimental.pallas import tpu_sc as plsc

_SIZE = 1.0
_RES = 64
_EMBED = 16
_TABLE = 2 ** 21
_N = 1048576

_NC = 2
_NS = 16
_NW = _NC * _NS
_L = 16

_PER_W = _N // _NW
_CHUNK = 1024
_NCHUNK = _PER_W // _CHUNK
_GSUB = 128
_NG = _CHUNK // _GSUB
_CBLK = _CHUNK // 128
_OBLK_W = _PER_W // 128

_NBLK = _TABLE // 128
_BLK_W = _NBLK // _NW
_RB = 8
_RCH = _BLK_W // _RB
_RENT = _RB * 128

_P2 = 2654435761
_P3 = 805459861


def _hash_tc_body(x_ref, y_ref, z_ref, o_ref):
    def quant(v):
        f = (v / _SIZE + 0.5) * _RES
        return jnp.minimum(f.astype(jnp.int32), _RES - 1)

    cx = quant(x_ref[...])
    cy = quant(y_ref[...])
    cz = quant(z_ref[...])
    h = cx ^ (cy * jnp.int32(_P2 - 2 ** 32)) ^ (cz * jnp.int32(_P3))
    o_ref[...] = h & jnp.int32(_TABLE - 1)


def _relayout_body(nt_hbm, out_hbm, buf0, buf1, rows0, rows1,
                   sem_in, sem_out):
    wid = lax.axis_index("s") * _NC + lax.axis_index("c")
    lane16 = lax.iota(jnp.int32, _L) * 16
    bufs = (buf0, buf1)
    rows = (rows0, rows1)

    def in_descs(par, c):
        b0 = wid * _BLK_W + c * _RB
        return [
            pltpu.make_async_copy(nt_hbm.at[e, pl.ds(b0, _RB)],
                                  bufs[par].at[e], sem_in)
            for e in range(2)
        ]

    def out_desc(par, c):
        ent0 = (wid * _BLK_W + c * _RB) * 128
        return pltpu.make_async_copy(
            rows[par], out_hbm.at[pl.ds(ent0 * _EMBED, _RENT * _EMBED)],
            sem_out)

    for d in in_descs(0, 0):
        d.start()

    @pl.loop(0, _RCH, step=2)
    def outer(i):
        for par in range(2):
            c = i + par
            for d in in_descs(par, c):
                d.wait()

            @pl.when(c + 1 < _RCH)
            def _():
                for d in in_descs(1 - par, c + 1):
                    d.start()

            @pl.when(c >= 2)
            def _():
                out_desc(par, c - 2).wait()

            for blk in range(_RB):
                vb = lane16 + blk * (128 * _EMBED)
                for e in range(2):
                    for fl in range(8):
                        f = 8 * e + fl
                        vals = [bufs[par][e, blk, fl, pl.ds(g * _L, _L)]
                                for g in range(8)]
                        for g in range(8):
                            plsc.store_scatter(
                                rows[par], [vb + (g * _L * _EMBED + f)],
                                vals[g])
            out_desc(par, c).start()

    for c in (_RCH - 2, _RCH - 1):
        out_desc(c % 2, c).wait()


def _gather_body(idx_hbm, table_hbm, out_hbm,
                 idx0, idx1, rv0, rv1, tr0, tr1,
                 gsem0, gsem1, osem0, osem1):
    wid = lax.axis_index("s") * _NC + lax.axis_index("c")
    base = wid * _PER_W
    lane = lax.iota(jnp.int32, _L)
    colf = [jnp.full((_L,), f, jnp.int32) for f in range(_EMBED)]
    idxs = (idx0, idx1)
    rvs = (rv0, rv1)
    trs = (tr0, tr1)
    gsems = (gsem0, gsem1)
    osems = (osem0, osem1)

    def g_descs(par):
        return [pltpu.make_async_copy(
            table_hbm.at[idxs[par].at[pl.ds(g * _GSUB, _GSUB)]],
            rvs[par].at[pl.ds(g * _GSUB, _GSUB)],
            gsems[par]) for g in range(_NG)]

    def o_descs(par, c):
        blk0 = wid * _OBLK_W + c * _CBLK
        return [pltpu.make_async_copy(
            trs[par].at[e], out_hbm.at[e, pl.ds(blk0, _CBLK)],
            osems[par]) for e in range(2)]

    pltpu.sync_copy(idx_hbm.at[pl.ds(base, _CHUNK)], idx0)
    for d in g_descs(0):
        d.start()

    @pl.loop(0, _NCHUNK, step=2)
    def outer(i):
        for par in range(2):
            c = i + par
            for d in g_descs(par):
                d.wait()

            @pl.when(c + 1 < _NCHUNK)
            def _():
                pltpu.sync_copy(
                    idx_hbm.at[pl.ds(base + (c + 1) * _CHUNK, _CHUNK)],
                    idxs[1 - par])
                for d in g_descs(1 - par):
                    d.start()

            @pl.when(c >= 2)
            def _():
                for d in o_descs(par, c - 2):
                    d.wait()

            for blk in range(_CBLK):
                vb = lane + blk * 128
                for e in range(2):
                    for fl in range(8):
                        f = 8 * e + fl
                        vals = [plsc.load_gather(rvs[par],
                                                 [vb + g * _L, colf[f]])
                                for g in range(8)]
                        for g in range(8):
                            trs[par][e, blk, fl, pl.ds(g * _L, _L)] = vals[g]

            for d in o_descs(par, c):
                d.start()

    for c in (_NCHUNK - 2, _NCHUNK - 1):
        for d in o_descs(c % 2, c):
            d.wait()


_SC_PARAMS = pltpu.CompilerParams(
    needs_layout_passes=False, use_tc_tiling_on_sc=False,
    disable_bounds_checks=True)


@jax.jit
def _lookup(xyz, table):
    x = xyz[:, 0].reshape(_N // 128, 128)
    y = xyz[:, 1].reshape(_N // 128, 128)
    z = xyz[:, 2].reshape(_N // 128, 128)

    idx2d = pl.pallas_call(
        _hash_tc_body,
        out_shape=jax.ShapeDtypeStruct((_N // 128, 128), jnp.int32),
    )(x, y, z)
    idx_flat = idx2d.reshape(-1)

    nt = table.T.reshape(2, 8, _NBLK, 128).transpose(0, 2, 1, 3)

    mesh = plsc.VectorSubcoreMesh(
        core_axis_name="c", subcore_axis_name="s",
        num_cores=_NC, num_subcores=_NS,
    )
    table_rm_flat = pl.kernel(
        _relayout_body,
        out_type=jax.ShapeDtypeStruct((_TABLE * _EMBED,), jnp.float32),
        mesh=mesh,
        scratch_types=[
            pltpu.VMEM((2, _RB, 8, 128), jnp.float32),
            pltpu.VMEM((2, _RB, 8, 128), jnp.float32),
            pltpu.VMEM((_RENT * _EMBED,), jnp.float32),
            pltpu.VMEM((_RENT * _EMBED,), jnp.float32),
            pltpu.SemaphoreType.DMA,
            pltpu.SemaphoreType.DMA,
        ],
        compiler_params=_SC_PARAMS,
    )(nt)
    table_rm = table_rm_flat.reshape(_TABLE, _EMBED)

    out4 = pl.kernel(
        _gather_body,
        out_type=jax.ShapeDtypeStruct((2, _N // 128, 8, 128), jnp.float32),
        mesh=mesh,
        scratch_types=[
            pltpu.VMEM((_CHUNK,), jnp.int32),
            pltpu.VMEM((_CHUNK,), jnp.int32),
            pltpu.VMEM((_CHUNK, _EMBED), jnp.float32),
            pltpu.VMEM((_CHUNK, _EMBED), jnp.float32),
            pltpu.VMEM((2, _CBLK, 8, 128), jnp.float32),
            pltpu.VMEM((2, _CBLK, 8, 128), jnp.float32),
            pltpu.SemaphoreType.DMA,
            pltpu.SemaphoreType.DMA,
            pltpu.SemaphoreType.DMA,
            pltpu.SemaphoreType.DMA,
        ],
        compiler_params=_SC_PARAMS,
    )(idx_flat, table_rm)

    return out4.transpose(1, 3, 0, 2).reshape(_N, _EMBED)


def kernel(xyz, table):
    return _lookup(xyz, table)


# speedup vs baseline: 293.4414x; 1.8684x over previous
import jax
import jax.numpy as jnp
from jax import lax
from jax.experimental import pallas as pl
from jax.experimental.pallas import tpu as pltpu
from jax.experimental.pallas import tpu_sc as plsc

_SIZE = 1.0
_RES = 64
_EMBED = 16
_TABLE = 2 ** 21
_N = 1048576

_NC = 2
_NS = 16
_NW = _NC * _NS
_L = 16

_PER_W = _N // _NW
_CHUNK = 512
_NCHUNK = _PER_W // _CHUNK
_GSUB = 128
_NG = _CHUNK // _GSUB
_CBLK = _CHUNK // 128
_OBLK_W = _PER_W // 128

_NBLK = _TABLE // 128
_BLK_W = _NBLK // _NW
_RB = 8
_RCH = _BLK_W // _RB
_RENT = _RB * 128

_LMASK = _L - 1

_P2 = 2654435761
_P3 = 805459861


def _hash_tc_body(x_ref, y_ref, z_ref, o_ref):
    def quant(v):
        f = (v / _SIZE + 0.5) * _RES
        return jnp.minimum(f.astype(jnp.int32), _RES - 1)

    cx = quant(x_ref[...])
    cy = quant(y_ref[...])
    cz = quant(z_ref[...])
    h = cx ^ (cy * jnp.int32(_P2 - 2 ** 32)) ^ (cz * jnp.int32(_P3))
    o_ref[...] = h & jnp.int32(_TABLE - 1)


def _relayout_body(nt_hbm, out_hbm, buf0, buf1, rows0, rows1,
                   sem_in, sem_out):
    wid = lax.axis_index("s") * _NC + lax.axis_index("c")
    lane = lax.iota(jnp.int32, _L)
    bufs = (buf0, buf1)
    rows = (rows0, rows1)

    def in_descs(par, c):
        b0 = wid * _BLK_W + c * _RB
        return [
            pltpu.make_async_copy(nt_hbm.at[e, pl.ds(b0, _RB)],
                                  bufs[par].at[e], sem_in)
            for e in range(2)
        ]

    def out_desc(par, c):
        ent0 = (wid * _BLK_W + c * _RB) * 128
        return pltpu.make_async_copy(
            rows[par], out_hbm.at[pl.ds(ent0 * _EMBED, _RENT * _EMBED)],
            sem_out)

    for d in in_descs(0, 0):
        d.start()

    @pl.loop(0, _RCH, step=2)
    def outer(i):
        for par in range(2):
            c = i + par
            for d in in_descs(par, c):
                d.wait()

            @pl.when(c + 1 < _RCH)
            def _():
                for d in in_descs(1 - par, c + 1):
                    d.start()

            @pl.when(c >= 2)
            def _():
                out_desc(par, c - 2).wait()

            @pl.loop(0, _RB)
            def blk_loop(blk):
                for e in range(2):
                    for fl in range(8):
                        f = 8 * e + fl
                        xbf = (lane * _EMBED
                               + jnp.bitwise_xor(lane, jnp.int32(f))
                               + blk * (128 * _EMBED))
                        vals = [bufs[par][e, blk, fl, pl.ds(g * _L, _L)]
                                for g in range(8)]
                        for g in range(8):
                            plsc.store_scatter(
                                rows[par],
                                [xbf + g * (_L * _EMBED)],
                                vals[g])
            out_desc(par, c).start()

    for c in (_RCH - 2, _RCH - 1):
        out_desc(c % 2, c).wait()


def _gather_body(idx_hbm, table_hbm, out_hbm,
                 idx0, idx1, rv0, rv1, tr0, tr1,
                 gsem0, gsem1, osem0, osem1):
    wid = lax.axis_index("s") * _NC + lax.axis_index("c")
    base = wid * _PER_W
    lane = lax.iota(jnp.int32, _L)
    idxs = (idx0, idx1)
    rvs = (rv0, rv1)
    trs = (tr0, tr1)
    gsems = (gsem0, gsem1)
    osems = (osem0, osem1)

    def g_descs(par):
        return [pltpu.make_async_copy(
            table_hbm.at[idxs[par].at[pl.ds(g * _GSUB, _GSUB)]],
            rvs[par].at[pl.ds(g * _GSUB, _GSUB)],
            gsems[par]) for g in range(_NG)]

    def o_descs(par, c):
        blk0 = wid * _OBLK_W + c * _CBLK
        return [pltpu.make_async_copy(
            trs[par].at[e], out_hbm.at[e, pl.ds(blk0, _CBLK)],
            osems[par]) for e in range(2)]

    pltpu.sync_copy(idx_hbm.at[pl.ds(base, _CHUNK)], idx0)
    for d in g_descs(0):
        d.start()

    @pl.loop(0, _NCHUNK, step=2)
    def outer(i):
        for par in range(2):
            c = i + par
            for d in g_descs(par):
                d.wait()

            @pl.when(c + 1 < _NCHUNK)
            def _():
                pltpu.sync_copy(
                    idx_hbm.at[pl.ds(base + (c + 1) * _CHUNK, _CHUNK)],
                    idxs[1 - par])
                for d in g_descs(1 - par):
                    d.start()

            @pl.when(c >= 2)
            def _():
                for d in o_descs(par, c - 2):
                    d.wait()

            for blk in range(_CBLK):
                vb = lane + blk * 128
                ms = [idxs[par][pl.ds(blk * 128 + g * _L, _L)] & _LMASK
                      for g in range(8)]
                for e in range(2):
                    for fl in range(8):
                        f = 8 * e + fl
                        vals = [plsc.load_gather(
                            rvs[par],
                            [vb + g * _L,
                             jnp.bitwise_xor(ms[g], jnp.int32(f))])
                                for g in range(8)]
                        for g in range(8):
                            trs[par][e, blk, fl, pl.ds(g * _L, _L)] = vals[g]

            for d in o_descs(par, c):
                d.start()

    for c in (_NCHUNK - 2, _NCHUNK - 1):
        for d in o_descs(c % 2, c):
            d.wait()


_SC_PARAMS = pltpu.CompilerParams(
    needs_layout_passes=False, use_tc_tiling_on_sc=False,
    disable_bounds_checks=True)


@jax.jit
def _lookup(xyz, table):
    x = xyz[:, 0].reshape(_N // 128, 128)
    y = xyz[:, 1].reshape(_N // 128, 128)
    z = xyz[:, 2].reshape(_N // 128, 128)

    idx2d = pl.pallas_call(
        _hash_tc_body,
        out_shape=jax.ShapeDtypeStruct((_N // 128, 128), jnp.int32),
    )(x, y, z)
    idx_flat = idx2d.reshape(-1)

    nt = table.T.reshape(2, 8, _NBLK, 128).transpose(0, 2, 1, 3)

    mesh = plsc.VectorSubcoreMesh(
        core_axis_name="c", subcore_axis_name="s",
        num_cores=_NC, num_subcores=_NS,
    )
    table_rm_flat = pl.kernel(
        _relayout_body,
        out_type=jax.ShapeDtypeStruct((_TABLE * _EMBED,), jnp.float32),
        mesh=mesh,
        scratch_types=[
            pltpu.VMEM((2, _RB, 8, 128), jnp.float32),
            pltpu.VMEM((2, _RB, 8, 128), jnp.float32),
            pltpu.VMEM((_RENT * _EMBED,), jnp.float32),
            pltpu.VMEM((_RENT * _EMBED,), jnp.float32),
            pltpu.SemaphoreType.DMA,
            pltpu.SemaphoreType.DMA,
        ],
        compiler_params=_SC_PARAMS,
    )(nt)
    table_rm = table_rm_flat.reshape(_TABLE, _EMBED)

    out4 = pl.kernel(
        _gather_body,
        out_type=jax.ShapeDtypeStruct((2, _N // 128, 8, 128), jnp.float32),
        mesh=mesh,
        scratch_types=[
            pltpu.VMEM((_CHUNK,), jnp.int32),
            pltpu.VMEM((_CHUNK,), jnp.int32),
            pltpu.VMEM((_CHUNK, _EMBED), jnp.float32),
            pltpu.VMEM((_CHUNK, _EMBED), jnp.float32),
            pltpu.VMEM((2, _CBLK, 8, 128), jnp.float32),
            pltpu.VMEM((2, _CBLK, 8, 128), jnp.float32),
            pltpu.SemaphoreType.DMA,
            pltpu.SemaphoreType.DMA,
            pltpu.SemaphoreType.DMA,
            pltpu.SemaphoreType.DMA,
        ],
        compiler_params=_SC_PARAMS,
    )(idx_flat, table_rm)

    return out4.transpose(1, 3, 0, 2).reshape(_N, _EMBED)


def kernel(xyz, table):
    return _lookup(xyz, table)


# speedup vs baseline: 388.8475x; 1.3251x over previous
import jax
import jax.numpy as jnp
from jax import lax
from jax.experimental import pallas as pl
from jax.experimental.pallas import tpu as pltpu
from jax.experimental.pallas import tpu_sc as plsc

_SIZE = 1.0
_RES = 64
_EMBED = 16
_TABLE = 2 ** 21
_N = 1048576

_NC = 2
_NS = 16
_NW = _NC * _NS
_L = 16

_PER_W = _N // _NW
_CHUNK = 1024
_NCHUNK = _PER_W // _CHUNK
_GSUB = 128
_NG = _CHUNK // _GSUB
_CBLK = _CHUNK // 128
_OBLK_W = _PER_W // 128

_NBLK = _TABLE // 128
_BLK_W = _NBLK // _NW
_RB = 8
_RCH = _BLK_W // _RB
_RENT = _RB * 128

_LMASK = _L - 1

_P2 = 2654435761
_P3 = 805459861


def _hash_tc_body(x_ref, y_ref, z_ref, o_ref):
    def quant(v):
        f = (v / _SIZE + 0.5) * _RES
        return jnp.minimum(f.astype(jnp.int32), _RES - 1)

    cx = quant(x_ref[...])
    cy = quant(y_ref[...])
    cz = quant(z_ref[...])
    h = cx ^ (cy * jnp.int32(_P2 - 2 ** 32)) ^ (cz * jnp.int32(_P3))
    o_ref[...] = h & jnp.int32(_TABLE - 1)


def _relayout_body(nt_hbm, out_hbm, buf0, buf1, rows0, rows1,
                   sem_in, sem_out):
    wid = lax.axis_index("s") * _NC + lax.axis_index("c")
    lane = lax.iota(jnp.int32, _L)
    bufs = (buf0, buf1)
    rows = (rows0, rows1)

    def in_descs(par, c):
        b0 = wid * _BLK_W + c * _RB
        return [
            pltpu.make_async_copy(nt_hbm.at[e, pl.ds(b0, _RB)],
                                  bufs[par].at[e], sem_in)
            for e in range(2)
        ]

    def out_desc(par, c):
        ent0 = (wid * _BLK_W + c * _RB) * 128
        return pltpu.make_async_copy(
            rows[par], out_hbm.at[pl.ds(ent0 * _EMBED, _RENT * _EMBED)],
            sem_out)

    for d in in_descs(0, 0):
        d.start()

    @pl.loop(0, _RCH, step=2)
    def outer(i):
        for par in range(2):
            c = i + par
            for d in in_descs(par, c):
                d.wait()

            @pl.when(c + 1 < _RCH)
            def _():
                for d in in_descs(1 - par, c + 1):
                    d.start()

            @pl.when(c >= 2)
            def _():
                out_desc(par, c - 2).wait()

            @pl.loop(0, _RB)
            def blk_loop(blk):
                for e in range(2):
                    for fl in range(8):
                        f = 8 * e + fl
                        xbf = (lane * _EMBED
                               + jnp.bitwise_xor(lane, jnp.int32(f))
                               + blk * (128 * _EMBED))
                        vals = [bufs[par][e, blk, fl, pl.ds(g * _L, _L)]
                                for g in range(8)]
                        for g in range(8):
                            plsc.store_scatter(
                                rows[par],
                                [xbf + g * (_L * _EMBED)],
                                vals[g])
            out_desc(par, c).start()

    for c in (_RCH - 2, _RCH - 1):
        out_desc(c % 2, c).wait()


def _gather_body(idx_hbm, table_hbm, out_hbm,
                 idx0, idx1, rv0, rv1, tr0, tr1,
                 gsem0, gsem1, osem0, osem1):
    wid = lax.axis_index("s") * _NC + lax.axis_index("c")
    base = wid * _PER_W
    lane = lax.iota(jnp.int32, _L)
    idxs = (idx0, idx1)
    rvs = (rv0, rv1)
    trs = (tr0, tr1)
    gsems = (gsem0, gsem1)
    osems = (osem0, osem1)

    def g_descs(par):
        return [pltpu.make_async_copy(
            table_hbm.at[idxs[par].at[pl.ds(g * _GSUB, _GSUB)]],
            rvs[par].at[pl.ds(g * _GSUB, _GSUB)],
            gsems[par]) for g in range(_NG)]

    def o_descs(par, c):
        blk0 = wid * _OBLK_W + c * _CBLK
        return [pltpu.make_async_copy(
            trs[par].at[e], out_hbm.at[e, pl.ds(blk0, _CBLK)],
            osems[par]) for e in range(2)]

    pltpu.sync_copy(idx_hbm.at[pl.ds(base, _CHUNK)], idx0)
    for d in g_descs(0):
        d.start()

    @pl.loop(0, _NCHUNK, step=2)
    def outer(i):
        for par in range(2):
            c = i + par
            for d in g_descs(par):
                d.wait()

            @pl.when(c + 1 < _NCHUNK)
            def _():
                pltpu.sync_copy(
                    idx_hbm.at[pl.ds(base + (c + 1) * _CHUNK, _CHUNK)],
                    idxs[1 - par])
                for d in g_descs(1 - par):
                    d.start()

            @pl.when(c >= 2)
            def _():
                for d in o_descs(par, c - 2):
                    d.wait()

            @pl.loop(0, _CBLK)
            def blk_loop(blk):
                vb = lane + blk * 128
                ms = [idxs[par][pl.ds(blk * 128 + g * _L, _L)] & _LMASK
                      for g in range(8)]
                for e in range(2):
                    for fl in range(8):
                        f = 8 * e + fl
                        vals = [plsc.load_gather(
                            rvs[par],
                            [vb + g * _L,
                             jnp.bitwise_xor(ms[g], jnp.int32(f))])
                                for g in range(8)]
                        for g in range(8):
                            trs[par][e, blk, fl, pl.ds(g * _L, _L)] = vals[g]

            for d in o_descs(par, c):
                d.start()

    for c in (_NCHUNK - 2, _NCHUNK - 1):
        for d in o_descs(c % 2, c):
            d.wait()


_SC_PARAMS = pltpu.CompilerParams(
    needs_layout_passes=False, use_tc_tiling_on_sc=False,
    disable_bounds_checks=True)


@jax.jit
def _lookup(xyz, table):
    x = xyz[:, 0].reshape(_N // 128, 128)
    y = xyz[:, 1].reshape(_N // 128, 128)
    z = xyz[:, 2].reshape(_N // 128, 128)

    idx2d = pl.pallas_call(
        _hash_tc_body,
        out_shape=jax.ShapeDtypeStruct((_N // 128, 128), jnp.int32),
    )(x, y, z)
    idx_flat = idx2d.reshape(-1)

    nt = table.T.reshape(2, 8, _NBLK, 128).transpose(0, 2, 1, 3)

    mesh = plsc.VectorSubcoreMesh(
        core_axis_name="c", subcore_axis_name="s",
        num_cores=_NC, num_subcores=_NS,
    )
    table_rm_flat = pl.kernel(
        _relayout_body,
        out_type=jax.ShapeDtypeStruct((_TABLE * _EMBED,), jnp.float32),
        mesh=mesh,
        scratch_types=[
            pltpu.VMEM((2, _RB, 8, 128), jnp.float32),
            pltpu.VMEM((2, _RB, 8, 128), jnp.float32),
            pltpu.VMEM((_RENT * _EMBED,), jnp.float32),
            pltpu.VMEM((_RENT * _EMBED,), jnp.float32),
            pltpu.SemaphoreType.DMA,
            pltpu.SemaphoreType.DMA,
        ],
        compiler_params=_SC_PARAMS,
    )(nt)
    table_rm = table_rm_flat.reshape(_TABLE, _EMBED)

    out4 = pl.kernel(
        _gather_body,
        out_type=jax.ShapeDtypeStruct((2, _N // 128, 8, 128), jnp.float32),
        mesh=mesh,
        scratch_types=[
            pltpu.VMEM((_CHUNK,), jnp.int32),
            pltpu.VMEM((_CHUNK,), jnp.int32),
            pltpu.VMEM((_CHUNK, _EMBED), jnp.float32),
            pltpu.VMEM((_CHUNK, _EMBED), jnp.float32),
            pltpu.VMEM((2, _CBLK, 8, 128), jnp.float32),
            pltpu.VMEM((2, _CBLK, 8, 128), jnp.float32),
            pltpu.SemaphoreType.DMA,
            pltpu.SemaphoreType.DMA,
            pltpu.SemaphoreType.DMA,
            pltpu.SemaphoreType.DMA,
        ],
        compiler_params=_SC_PARAMS,
    )(idx_flat, table_rm)

    return out4.transpose(1, 3, 0, 2).reshape(_N, _EMBED)


def kernel(xyz, table):
    return _lookup(xyz, table)
